# baseline (speedup 1.0000x reference)
import functools

import jax
import jax.numpy as jnp
from jax import lax
from jax.experimental import pallas as pl
from jax.experimental.pallas import tpu as pltpu
from jax.experimental.pallas import tpu_sc as plsc

_L = 128
_COLS = (0, 1, 3, 7, 15, 31, 50)
_NBUD = 51


def _bitonic_stage(arrs, K, j, m, log_r):
    key, idx = arrs[0], arrs[1]
    if j >= log_r:
        axis, sh = 1, 1 << (j - log_r)
    else:
        axis, sh = 0, 1 << j
    pos = jax.lax.broadcasted_iota(jnp.int32, key.shape, axis)
    upper = (pos & sh) != 0
    dim = key.shape[axis]

    def partner(a):
        return jnp.where(upper, pltpu.roll(a, sh, axis=axis),
                         pltpu.roll(a, dim - sh, axis=axis))

    pk, pi = partner(key), partner(idx)
    gt = (key > pk) | ((key == pk) & (idx > pi))
    take = gt ^ upper
    if K < m:
        if K < log_r:
            dpos = jax.lax.broadcasted_iota(jnp.int32, key.shape, 0)
            desc = ((dpos >> K) & 1) != 0
        else:
            dpos = jax.lax.broadcasted_iota(jnp.int32, key.shape, 1)
            desc = ((dpos >> (K - log_r)) & 1) != 0
        take = take ^ desc
    out = [jnp.where(take, pk, key), jnp.where(take, pi, idx)]
    for a in arrs[2:]:
        out.append(jnp.where(take, partner(a), a))
    return out


def _sort_kernel(n_arr, m, log_r, *refs):
    arrs = [r[...] for r in refs[:n_arr]]
    for K in range(1, m + 1):
        for j in range(K - 1, -1, -1):
            arrs = _bitonic_stage(arrs, K, j, m, log_r)
    outs = refs[n_arr:]
    outs[0][...] = arrs[1]
    for o, a in zip(outs[1:], arrs[2:]):
        o[...] = a


def _sorted_idx(arrs):
    rr = arrs[0].shape[0]
    log_r = rr.bit_length() - 1
    m = log_r + 7
    out_shape = [jax.ShapeDtypeStruct((rr, _L), jnp.int32)]
    out_shape += [jax.ShapeDtypeStruct((rr, _L), a.dtype) for a in arrs[2:]]
    fn = pl.pallas_call(
        functools.partial(_sort_kernel, len(arrs), m, log_r),
        out_shape=out_shape,
    )
    return fn(*arrs)


def _make_sc_gather(m_elems, col):
    per = m_elems // 32
    mesh = plsc.VectorSubcoreMesh(core_axis_name="c", subcore_axis_name="s")

    @functools.partial(
        pl.kernel, mesh=mesh,
        out_type=jax.ShapeDtypeStruct((m_elems,), jnp.float32),
        scratch_types=[
            pltpu.VMEM((per,), jnp.int32),
            pltpu.VMEM((per,), jnp.int32),
            pltpu.VMEM((per,), jnp.float32),
            pltpu.SemaphoreType.DMA,
        ],
    )
    def g(table_hbm, idx_hbm, out_hbm, idx_v, scaled_v, vals_v, sem):
        wid = lax.axis_index("s") * 2 + lax.axis_index("c")
        base = wid * per
        pltpu.sync_copy(idx_hbm.at[pl.ds(base, per)], idx_v)

        def body(i, carry):
            sl = pl.ds(i * 16, 16)
            scaled_v[sl] = idx_v[sl] * _NBUD + col
            return carry

        lax.fori_loop(0, per // 16, body, 0)
        pltpu.async_copy(table_hbm.at[scaled_v], vals_v, sem).wait()
        pltpu.sync_copy(vals_v, out_hbm.at[pl.ds(base, per)])

    return g


def _make_sc_gather2(m_elems, c1, c2):
    per = m_elems // 32
    mesh = plsc.VectorSubcoreMesh(core_axis_name="c", subcore_axis_name="s")

    @functools.partial(
        pl.kernel, mesh=mesh,
        out_type=[jax.ShapeDtypeStruct((m_elems,), jnp.float32),
                  jax.ShapeDtypeStruct((m_elems,), jnp.float32)],
        scratch_types=[
            pltpu.VMEM((per,), jnp.int32),
            pltpu.VMEM((per,), jnp.int32),
            pltpu.VMEM((per,), jnp.int32),
            pltpu.VMEM((per,), jnp.float32),
            pltpu.VMEM((per,), jnp.float32),
            pltpu.SemaphoreType.DMA,
            pltpu.SemaphoreType.DMA,
        ],
    )
    def g(table_hbm, idx_hbm, o1_hbm, o2_hbm, idx_v, s1_v, s2_v, v1_v, v2_v,
          sem1, sem2):
        wid = lax.axis_index("s") * 2 + lax.axis_index("c")
        base = wid * per
        pltpu.sync_copy(idx_hbm.at[pl.ds(base, per)], idx_v)

        def body(i, carry):
            sl = pl.ds(i * 16, 16)
            scaled = idx_v[sl] * _NBUD
            s1_v[sl] = scaled + c1
            s2_v[sl] = scaled + c2
            return carry

        lax.fori_loop(0, per // 16, body, 0)
        cp1 = pltpu.async_copy(table_hbm.at[s1_v], v1_v, sem1)
        cp2 = pltpu.async_copy(table_hbm.at[s2_v], v2_v, sem2)
        cp1.wait()
        cp2.wait()
        pltpu.sync_copy(v1_v, o1_hbm.at[pl.ds(base, per)])
        pltpu.sync_copy(v2_v, o2_hbm.at[pl.ds(base, per)])

    return g


def kernel(learning_curves, mask):
    del mask
    lc = learning_curves[0]
    table = learning_curves.reshape(-1)
    n = lc.shape[0]

    idx = jnp.arange(n, dtype=jnp.int32)
    cur = [lc[:, _COLS[0]].reshape(-1, _L), idx.reshape(-1, _L)]
    parts = []
    for r in range(7):
        srt = _sorted_idx(cur)
        sidx = srt[0]
        if r == 6:
            parts.append(sidx.T.reshape(-1).astype(jnp.float32))
            break
        parts.append(sidx[:, :64].T.reshape(-1).astype(jnp.float32))
        surv_flat = sidx[:, 64:].reshape(-1)
        nxt_idx = surv_flat.reshape(-1, _L)
        if r % 2 == 0:
            v1, v2 = _make_sc_gather2(
                surv_flat.shape[0], _COLS[r + 1], _COLS[r + 2])(table, surv_flat)
            cur = [v1.reshape(-1, _L), nxt_idx, v2.reshape(-1, _L)]
        else:
            nxt_key = srt[1][:, 64:].reshape(-1, _L)
            cur = [nxt_key, nxt_idx]
    return jnp.concatenate(parts)


# speedup vs baseline: 1.0015x; 1.0015x over previous
import functools

import jax
import jax.numpy as jnp
from jax import lax
from jax.experimental import pallas as pl
from jax.experimental.pallas import tpu as pltpu
from jax.experimental.pallas import tpu_sc as plsc

_L = 128
_COLS = (0, 1, 3, 7, 15, 31, 50)
_NBUD = 51


def _bitonic_stage(arrs, K, j, m, log_r):
    key, idx = arrs[0], arrs[1]
    if j >= log_r:
        axis, sh = 1, 1 << (j - log_r)
    else:
        axis, sh = 0, 1 << j
    pos = jax.lax.broadcasted_iota(jnp.int32, key.shape, axis)
    upper = (pos & sh) != 0
    dim = key.shape[axis]

    def partner(a):
        return jnp.where(upper, pltpu.roll(a, sh, axis=axis),
                         pltpu.roll(a, dim - sh, axis=axis))

    pk, pi = partner(key), partner(idx)
    gt = (key > pk) | ((key == pk) & (idx > pi))
    take = gt ^ upper
    if K < m:
        if K < log_r:
            dpos = jax.lax.broadcasted_iota(jnp.int32, key.shape, 0)
            desc = ((dpos >> K) & 1) != 0
        else:
            dpos = jax.lax.broadcasted_iota(jnp.int32, key.shape, 1)
            desc = ((dpos >> (K - log_r)) & 1) != 0
        take = take ^ desc
    out = [jnp.where(take, pk, key), jnp.where(take, pi, idx)]
    for a in arrs[2:]:
        out.append(jnp.where(take, partner(a), a))
    return out


def _sort_kernel(n_arr, m, log_r, *refs):
    arrs = [r[...] for r in refs[:n_arr]]
    for K in range(1, m + 1):
        for j in range(K - 1, -1, -1):
            arrs = _bitonic_stage(arrs, K, j, m, log_r)
    refs[n_arr][...] = arrs[1]


def _sorted_idx(arrs):
    rr = arrs[0].shape[0]
    log_r = rr.bit_length() - 1
    m = log_r + 7
    fn = pl.pallas_call(
        functools.partial(_sort_kernel, len(arrs), m, log_r),
        out_shape=jax.ShapeDtypeStruct((rr, _L), jnp.int32),
    )
    return fn(*arrs)


def _make_sc_gather(m_elems, col):
    per = m_elems // 32
    mesh = plsc.VectorSubcoreMesh(core_axis_name="c", subcore_axis_name="s")

    @functools.partial(
        pl.kernel, mesh=mesh,
        out_type=jax.ShapeDtypeStruct((m_elems,), jnp.float32),
        scratch_types=[
            pltpu.VMEM((per,), jnp.int32),
            pltpu.VMEM((per,), jnp.int32),
            pltpu.VMEM((per,), jnp.float32),
            pltpu.SemaphoreType.DMA,
        ],
    )
    def g(table_hbm, idx_hbm, out_hbm, idx_v, scaled_v, vals_v, sem):
        wid = lax.axis_index("s") * 2 + lax.axis_index("c")
        base = wid * per
        pltpu.sync_copy(idx_hbm.at[pl.ds(base, per)], idx_v)

        def body(i, carry):
            sl = pl.ds(i * 16, 16)
            scaled_v[sl] = idx_v[sl] * _NBUD + col
            return carry

        lax.fori_loop(0, per // 16, body, 0)
        pltpu.async_copy(table_hbm.at[scaled_v], vals_v, sem).wait()
        pltpu.sync_copy(vals_v, out_hbm.at[pl.ds(base, per)])

    return g


def kernel(learning_curves, mask):
    del mask
    lc = learning_curves[0]
    table = learning_curves.reshape(-1)
    n = lc.shape[0]

    idx = jnp.arange(n, dtype=jnp.int32)
    cur_key = lc[:, _COLS[0]].reshape(-1, _L)
    cur_idx = idx.reshape(-1, _L)
    parts = []
    for r in range(7):
        sidx = _sorted_idx([cur_key, cur_idx])
        if r < 6:
            parts.append(sidx[:, :64].T.reshape(-1).astype(jnp.float32))
            surv_flat = sidx[:, 64:].reshape(-1)
            vals = _make_sc_gather(surv_flat.shape[0], _COLS[r + 1])(table, surv_flat)
            cur_idx = surv_flat.reshape(-1, _L)
            cur_key = vals.reshape(-1, _L)
        else:
            parts.append(sidx.T.reshape(-1).astype(jnp.float32))
    return jnp.concatenate(parts)
